# baseline (speedup 1.0000x reference)
import functools

import jax
import jax.numpy as jnp
from jax import lax
from jax.experimental import pallas as pl
from jax.experimental.pallas import tpu as pltpu
from jax.experimental.pallas import tpu_sc as plsc

_NC = 2
_NS = 16
_NW = _NC * _NS
_LANES = 16

_RND = 12582912.0

_CH = 256
_NB = 4
_SEQP = 256


def _quantize(xv):
    v = jnp.minimum(jnp.maximum(xv, -65.0), 65.0)
    rr = (v + _RND) - _RND
    t = rr.astype(jnp.int32)
    t = jnp.minimum(jnp.maximum(t, -64), 64) + 65
    t = jnp.where(xv == jnp.inf, 130, t)
    t = jnp.where(xv == -jnp.inf, 0, t)
    t = jnp.where(xv != xv, 0, t)
    return t


def _make_tc_quantize(nrows, seq, blk):
    def body(x_ref, o_ref):
        t = _quantize(x_ref[...])
        o_ref[...] = jnp.concatenate(
            [t, jnp.zeros((blk, _SEQP - seq), jnp.int32)], axis=1
        )

    return pl.pallas_call(
        body,
        grid=(nrows // blk,),
        in_specs=[pl.BlockSpec((blk, seq), lambda i: (i, 0))],
        out_specs=pl.BlockSpec((blk, _SEQP), lambda i: (i, 0)),
        out_shape=jax.ShapeDtypeStruct((nrows, _SEQP), jnp.int32),
    )


def _make_sc_lookup(nrows, seq, nvoc, D):
    rows_per_w = nrows // _NW
    per = rows_per_w * seq
    nch = per // _CH
    ngrp = -(-seq // _LANES)
    n = nrows * seq
    mesh = plsc.VectorSubcoreMesh(core_axis_name="c", subcore_axis_name="s")

    @functools.partial(
        pl.kernel,
        mesh=mesh,
        out_type=jax.ShapeDtypeStruct((n, D), jnp.float32),
        scratch_types=[
            pltpu.VMEM((rows_per_w, _SEQP), jnp.int32),
            pltpu.VMEM((per + _LANES,), jnp.int32),
            pltpu.VMEM((_NB, _CH, D), jnp.float32),
        ]
        + [pltpu.SemaphoreType.DMA] * (2 * _NB),
        compiler_params=pltpu.CompilerParams(use_tc_tiling_on_sc=False),
    )
    def run(pid_hbm, tab_hbm, out_hbm, pad_v, idx_v, buf, *sems):
        gsem, wsem = sems[:_NB], sems[_NB:]
        wid = lax.axis_index("s") * _NC + lax.axis_index("c")
        row0 = wid * per
        tab_off = wid * nvoc
        pltpu.sync_copy(pid_hbm.at[pl.ds(wid * rows_per_w, rows_per_w)], pad_v)

        def row(r, carry):
            for g in range(ngrp):
                vec = pad_v[r, pl.ds(g * _LANES, _LANES)]
                idx_v[pl.ds(r * seq + g * _LANES, _LANES)] = vec + tab_off
            return carry

        lax.fori_loop(0, rows_per_w, row, 0)

        def gather(c):
            b = c % _NB
            return [
                pltpu.async_copy(
                    tab_hbm.at[idx_v.at[pl.ds(c * _CH + j * 128, 128)]],
                    buf.at[b, pl.ds(j * 128, 128)],
                    gsem[b],
                )
                for j in range(_CH // 128)
            ]

        gathers = {c: gather(c) for c in range(2)}
        writes = {}
        for c in range(nch):
            b = c % _NB
            if c + 2 < nch:
                if c - 2 >= 0:
                    writes.pop(c - 2).wait()
                gathers[c + 2] = gather(c + 2)
            for cp in gathers.pop(c):
                cp.wait()
            writes[c] = pltpu.async_copy(
                buf.at[b], out_hbm.at[pl.ds(row0 + c * _CH, _CH)], wsem[b]
            )
        for c in sorted(writes):
            writes.pop(c).wait()

    return run


def kernel(x, table):
    b, seq = x.shape[0], x.shape[1]
    D = table.shape[1]
    ids = _make_tc_quantize(b, seq, blk=512)(x.reshape(b, seq))
    out = _make_sc_lookup(b, seq, table.shape[0], D)(
        ids, jnp.tile(table, (_NW, 1))
    )
    return out.reshape(b, seq, D)


# speedup vs baseline: 1.7504x; 1.7504x over previous
import functools

import jax
import jax.numpy as jnp
from jax import lax
from jax.experimental import pallas as pl
from jax.experimental.pallas import tpu as pltpu
from jax.experimental.pallas import tpu_sc as plsc

_NC = 2
_NS = 16
_NW = _NC * _NS
_LANES = 16

_RND = 12582912.0

_CH = 256
_NB = 4


def _make_sc_lookup(n, nvoc, D):
    per = n // _NW
    nch = per // _CH
    mesh = plsc.VectorSubcoreMesh(core_axis_name="c", subcore_axis_name="s")

    @functools.partial(
        pl.kernel,
        mesh=mesh,
        out_type=jax.ShapeDtypeStruct((n, 2 * D), jnp.float32),
        scratch_types=[
            pltpu.VMEM((per,), jnp.float32),
            pltpu.VMEM((per,), jnp.int32),
            pltpu.VMEM((_NB, _CH, D), jnp.float32),
        ]
        + [pltpu.SemaphoreType.DMA] * (2 * _NB),
        compiler_params=pltpu.CompilerParams(use_tc_tiling_on_sc=False),
    )
    def run(x_hbm, tab_hbm, out_hbm, x_v, idx_v, buf, *sems):
        gsem, wsem = sems[:_NB], sems[_NB:]
        wid = lax.axis_index("s") * _NC + lax.axis_index("c")
        row0 = wid * per
        tab_off = wid * nvoc
        pltpu.sync_copy(x_hbm.at[pl.ds(row0, per)], x_v)

        def grp(g, carry):
            xv = x_v[pl.ds(g * _LANES, _LANES)]
            v = jnp.minimum(jnp.maximum(xv, -65.0), 65.0)
            rr = (v + _RND) - _RND
            t = rr.astype(jnp.int32)
            t = jnp.minimum(jnp.maximum(t, -64), 64) + 65
            t = jnp.where(xv == jnp.inf, 130, t)
            t = jnp.where(xv == -jnp.inf, 0, t)
            t = jnp.where(xv != xv, 0, t)
            idx_v[pl.ds(g * _LANES, _LANES)] = t + tab_off
            return carry

        lax.fori_loop(0, per // _LANES, grp, 0)

        def gather(c):
            b = c % _NB
            return [
                pltpu.async_copy(
                    tab_hbm.at[idx_v.at[pl.ds(c * _CH + j * 128, 128)]],
                    buf.at[b, pl.ds(j * 128, 128)],
                    gsem[b],
                )
                for j in range(_CH // 128)
            ]

        gathers = {c: gather(c) for c in range(2)}
        writes = {}
        for c in range(nch):
            b = c % _NB
            if c + 2 < nch:
                if c - 2 >= 0:
                    writes.pop(c - 2).wait()
                gathers[c + 2] = gather(c + 2)
            for cp in gathers.pop(c):
                cp.wait()
            writes[c] = pltpu.async_copy(
                buf.at[b],
                out_hbm.at[pl.ds(row0 + c * _CH, _CH), pl.ds(0, D)],
                wsem[b],
            )
        for c in sorted(writes):
            writes.pop(c).wait()

    return run


def kernel(x, table):
    b, seq = x.shape[0], x.shape[1]
    D = table.shape[1]
    n = b * seq
    out = _make_sc_lookup(n, table.shape[0], D)(
        x.reshape(n), jnp.tile(table, (_NW, 1))
    )
    return out[:, :D].reshape(b, seq, D)
